# baseline (speedup 1.0000x reference)
import functools

import jax
import jax.numpy as jnp
from jax import lax
from jax.experimental import pallas as pl
from jax.experimental.pallas import tpu as pltpu
from jax.experimental.pallas import tpu_sc as plsc

D = 128
B = 4096
L = 50
N_TOK = B * L

SEG = 2
B_SEG = B // SEG
TOK_SEG = B_SEG * L

NC = 2
NS = 16
NW = NC * NS
CHUNK = 128
CH_W = TOK_SEG // (NW * CHUNK)
assert CH_W * NW * CHUNK == TOK_SEG
CH_PAD = (CH_W + 7) // 8 * 8

NBUF = 4


def _sc_gather_body(table_hbm, idx_hbm, out_hbm, idx_v, rows_v, gsem, ssem):
    wid = lax.axis_index("s") * NC + lax.axis_index("c")
    row0 = wid * CH_W

    pltpu.sync_copy(idx_hbm.at[pl.ds(wid * CH_PAD, CH_PAD)], idx_v)

    def gather(j):
        return pltpu.async_copy(
            table_hbm.at[idx_v.at[j]], rows_v.at[j % NBUF], gsem)

    def gather_wait(j):
        pltpu.make_async_copy(
            table_hbm.at[idx_v.at[j]], rows_v.at[j % NBUF], gsem).wait()

    def store(j):
        return pltpu.async_copy(
            rows_v.at[j % NBUF], out_hbm.at[pl.ds((row0 + j) * CHUNK, CHUNK)],
            ssem)

    def store_wait(j):
        pltpu.make_async_copy(
            rows_v.at[j % NBUF], out_hbm.at[pl.ds((row0 + j) * CHUNK, CHUNK)],
            ssem).wait()

    for j in range(NBUF - 1):
        gather(j)

    def body(j, carry):
        gather_wait(j)
        store(j)

        @pl.when(j >= 1)
        def _():
            store_wait(j - 1)

        @pl.when(j + NBUF - 1 < CH_W)
        def _():
            gather(j + NBUF - 1)

        return carry

    lax.fori_loop(0, CH_W, body, 0)
    store_wait(CH_W - 1)


@functools.cache
def _sc_gather():
    mesh = plsc.VectorSubcoreMesh(core_axis_name="c", subcore_axis_name="s",
                                  num_cores=NC, num_subcores=NS)
    return pl.kernel(
        _sc_gather_body,
        out_type=jax.ShapeDtypeStruct((TOK_SEG, D), jnp.float32),
        mesh=mesh,
        scratch_types=[
            pltpu.VMEM((CH_PAD, CHUNK), jnp.int32),
            pltpu.VMEM((NBUF, CHUNK, D), jnp.float32),
            pltpu.SemaphoreType.DMA,
            pltpu.SemaphoreType.DMA,
        ],
    )

BA = 64
TC_BLOCK = BA * L
N_BLOCKS_SEG = B_SEG // BA


def _tc_body(x_ref, g_ref, wa_ref, ba_ref, wt_ref, bt_ref, scale_ref,
             out_ref, m_ref, uv_ref):
    @pl.when(pl.program_id(0) == 0)
    def _():
        wa = wa_ref[...]
        wt = wt_ref[...]
        m_ref[...] = lax.dot_general(
            wa, wt, (((0,), (0,)), ((), ())),
            preferred_element_type=jnp.float32)
        uv_ref[0:1, :] = lax.dot_general(
            bt_ref[...], wa, (((1,), (0,)), ((), ())),
            preferred_element_type=jnp.float32)
        uv_ref[1:2, :] = lax.dot_general(
            ba_ref[...], wt, (((1,), (0,)), ((), ())),
            preferred_element_type=jnp.float32)

    x = x_ref[...].reshape(TC_BLOCK, D)
    g = g_ref[...]
    z = lax.dot_general(x, m_ref[...], (((1,), (0,)), ((), ())),
                        preferred_element_type=jnp.float32)
    z = z + uv_ref[1:2, :]
    dots = jnp.sum(z * g, axis=1)
    xu = jnp.sum(x * uv_ref[0:1, :], axis=1)
    c = jnp.sum(ba_ref[...] * bt_ref[...])
    out_ref[...] = ((dots + xu + c) * scale_ref[0, 0]).reshape(BA, L)


def _tc_call(seg, x, g, Wa, ba2, Wt, bt2, scale2):
    blk0 = seg * N_BLOCKS_SEG
    return pl.pallas_call(
        _tc_body,
        grid=(N_BLOCKS_SEG,),
        in_specs=[
            pl.BlockSpec((BA, L, D), lambda i: (blk0 + i, 0, 0)),
            pl.BlockSpec((TC_BLOCK, D), lambda i: (i, 0)),
            pl.BlockSpec((D, D), lambda i: (0, 0)),
            pl.BlockSpec((1, D), lambda i: (0, 0)),
            pl.BlockSpec((D, D), lambda i: (0, 0)),
            pl.BlockSpec((1, D), lambda i: (0, 0)),
            pl.BlockSpec(memory_space=pltpu.SMEM),
        ],
        out_specs=pl.BlockSpec((BA, L), lambda i: (i, 0)),
        out_shape=jax.ShapeDtypeStruct((B_SEG, L), jnp.float32),
        scratch_shapes=[
            pltpu.VMEM((D, D), jnp.float32),
            pltpu.VMEM((2, D), jnp.float32),
        ],
    )(x, g, Wa, ba2, Wt, bt2, scale2)


def kernel(actor_emb, topic_ids, Wa, ba, table, Wt, bt, scale):
    idx4d = topic_ids.reshape(SEG, NW, CH_W, CHUNK).astype(jnp.int32)
    idx_pad = jnp.pad(idx4d, ((0, 0), (0, 0), (0, CH_PAD - CH_W), (0, 0)))
    idx_pad = idx_pad.reshape(SEG, NW * CH_PAD, CHUNK)

    ba2 = ba.reshape(1, D)
    bt2 = bt.reshape(1, D)
    scale2 = scale.reshape(1, 1)

    outs = []
    for s in range(SEG):
        g = _sc_gather()(table, idx_pad[s])
        outs.append(_tc_call(s, actor_emb, g, Wa, ba2, Wt, bt2, scale2))
    return jnp.concatenate(outs, axis=0)


# speedup vs baseline: 1.6910x; 1.6910x over previous
import functools

import jax
import jax.numpy as jnp
from jax import lax
from jax.experimental import pallas as pl
from jax.experimental.pallas import tpu as pltpu
from jax.experimental.pallas import tpu_sc as plsc

D = 128
B = 4096
L = 50
N_TOK = B * L

NC = 2
NS = 16
NW = NC * NS
CHUNK = 128
CH_W = N_TOK // (NW * CHUNK)
CH_PAD = (CH_W + 7) // 8 * 8

NBUF = 4


def _sc_gather_body(table_hbm, idx_hbm, out_hbm, idx_v, rows_v, gsem, ssem):
    wid = lax.axis_index("s") * NC + lax.axis_index("c")
    row0 = wid * CH_W

    pltpu.sync_copy(idx_hbm.at[pl.ds(wid * CH_PAD, CH_PAD)], idx_v)

    def gather(j):
        return pltpu.async_copy(
            table_hbm.at[idx_v.at[j]], rows_v.at[j % NBUF], gsem)

    def gather_wait(j):
        pltpu.make_async_copy(
            table_hbm.at[idx_v.at[j]], rows_v.at[j % NBUF], gsem).wait()

    def store(j):
        return pltpu.async_copy(
            rows_v.at[j % NBUF], out_hbm.at[pl.ds((row0 + j) * CHUNK, CHUNK)],
            ssem)

    def store_wait(j):
        pltpu.make_async_copy(
            rows_v.at[j % NBUF], out_hbm.at[pl.ds((row0 + j) * CHUNK, CHUNK)],
            ssem).wait()

    for j in range(NBUF - 1):
        gather(j)

    def body(j, carry):
        gather_wait(j)
        store(j)

        @pl.when(j >= 1)
        def _():
            store_wait(j - 1)

        @pl.when(j + NBUF - 1 < CH_W)
        def _():
            gather(j + NBUF - 1)

        return carry

    lax.fori_loop(0, CH_W, body, 0)
    store_wait(CH_W - 1)


@functools.cache
def _sc_gather():
    mesh = plsc.VectorSubcoreMesh(core_axis_name="c", subcore_axis_name="s",
                                  num_cores=NC, num_subcores=NS)
    return pl.kernel(
        _sc_gather_body,
        out_type=jax.ShapeDtypeStruct((N_TOK, D), jnp.float32),
        mesh=mesh,
        scratch_types=[
            pltpu.VMEM((CH_PAD, CHUNK), jnp.int32),
            pltpu.VMEM((NBUF, CHUNK, D), jnp.float32),
            pltpu.SemaphoreType.DMA,
            pltpu.SemaphoreType.DMA,
        ],
    )

L_BLK = 10
A_BLK = 1024
TC_BLOCK = L_BLK * A_BLK


def _tc_body(x_ref, g_ref, wa_ref, ba_ref, wt_ref, bt_ref, scale_ref,
             out_ref, m_ref, uv_ref):
    @pl.when((pl.program_id(0) == 0) & (pl.program_id(1) == 0))
    def _():
        wa = wa_ref[...]
        wt = wt_ref[...]
        m_ref[...] = lax.dot_general(
            wa, wt, (((0,), (0,)), ((), ())),
            preferred_element_type=jnp.float32)
        uv_ref[0:1, :] = lax.dot_general(
            bt_ref[...], wa, (((1,), (0,)), ((), ())),
            preferred_element_type=jnp.float32)
        uv_ref[1:2, :] = lax.dot_general(
            ba_ref[...], wt, (((1,), (0,)), ((), ())),
            preferred_element_type=jnp.float32)

    x = x_ref[...].reshape(TC_BLOCK, D)
    g = g_ref[...].reshape(TC_BLOCK, D)
    z = lax.dot_general(x, m_ref[...], (((1,), (0,)), ((), ())),
                        preferred_element_type=jnp.float32)
    z = z + uv_ref[1:2, :]
    dots = jnp.sum(z * g, axis=1)
    xu = jnp.sum(x * uv_ref[0:1, :], axis=1)
    c = jnp.sum(ba_ref[...] * bt_ref[...])
    out_ref[...] = ((dots + xu + c) * scale_ref[0, 0]).reshape(
        L_BLK, A_BLK // 128, 128)


def _tc_call(x3, g3, Wa, ba2, Wt, bt2, scale2):
    return pl.pallas_call(
        _tc_body,
        grid=(L // L_BLK, B // A_BLK),
        in_specs=[
            pl.BlockSpec((L_BLK, A_BLK, D), lambda i, j: (i, j, 0)),
            pl.BlockSpec((L_BLK, A_BLK, D), lambda i, j: (i, j, 0)),
            pl.BlockSpec((D, D), lambda i, j: (0, 0)),
            pl.BlockSpec((1, D), lambda i, j: (0, 0)),
            pl.BlockSpec((D, D), lambda i, j: (0, 0)),
            pl.BlockSpec((1, D), lambda i, j: (0, 0)),
            pl.BlockSpec(memory_space=pltpu.SMEM),
        ],
        out_specs=pl.BlockSpec((L_BLK, A_BLK // 128, 128),
                               lambda i, j: (i, j, 0)),
        out_shape=jax.ShapeDtypeStruct((L, B // 128, 128), jnp.float32),
        scratch_shapes=[
            pltpu.VMEM((D, D), jnp.float32),
            pltpu.VMEM((2, D), jnp.float32),
        ],
    )(x3, g3, Wa, ba2, Wt, bt2, scale2)


def kernel(actor_emb, topic_ids, Wa, ba, table, Wt, bt, scale):
    x3 = jnp.transpose(actor_emb, (1, 0, 2))
    ids_lm = jnp.transpose(topic_ids, (1, 0)).astype(jnp.int32)

    idx3d = ids_lm.reshape(NW, CH_W, CHUNK)
    idx_pad = jnp.pad(idx3d, ((0, 0), (0, CH_PAD - CH_W), (0, 0)))
    g = _sc_gather()(table, idx_pad.reshape(NW * CH_PAD, CHUNK))
    g3 = g.reshape(L, B, D)

    out3 = _tc_call(x3, g3, Wa, ba.reshape(1, D), Wt, bt.reshape(1, D),
                    scale.reshape(1, 1))
    return jnp.transpose(out3.reshape(L, B), (1, 0))


# speedup vs baseline: 1.7055x; 1.0086x over previous
import functools

import jax
import jax.numpy as jnp
from jax import lax
from jax.experimental import pallas as pl
from jax.experimental.pallas import tpu as pltpu
from jax.experimental.pallas import tpu_sc as plsc

D = 128
B = 4096
L = 50
N_TOK = B * L

SEG = 2
L_SEG = L // SEG
TOK_SEG = L_SEG * B

NC = 2
NS = 16
NW = NC * NS
CHUNK = 128
CH_W = TOK_SEG // (NW * CHUNK)
assert CH_W * NW * CHUNK == TOK_SEG
CH_PAD = (CH_W + 7) // 8 * 8

NBUF = 4


def _sc_gather_body(table_hbm, idx_hbm, out_hbm, idx_v, rows_v, gsem, ssem):
    wid = lax.axis_index("s") * NC + lax.axis_index("c")
    row0 = wid * CH_W

    pltpu.sync_copy(idx_hbm.at[pl.ds(wid * CH_PAD, CH_PAD)], idx_v)

    def gather(j):
        return pltpu.async_copy(
            table_hbm.at[idx_v.at[j]], rows_v.at[j % NBUF], gsem)

    def gather_wait(j):
        pltpu.make_async_copy(
            table_hbm.at[idx_v.at[j]], rows_v.at[j % NBUF], gsem).wait()

    def store(j):
        return pltpu.async_copy(
            rows_v.at[j % NBUF], out_hbm.at[pl.ds((row0 + j) * CHUNK, CHUNK)],
            ssem)

    def store_wait(j):
        pltpu.make_async_copy(
            rows_v.at[j % NBUF], out_hbm.at[pl.ds((row0 + j) * CHUNK, CHUNK)],
            ssem).wait()

    for j in range(NBUF - 1):
        gather(j)

    def body(j, carry):
        gather_wait(j)
        store(j)

        @pl.when(j >= 1)
        def _():
            store_wait(j - 1)

        @pl.when(j + NBUF - 1 < CH_W)
        def _():
            gather(j + NBUF - 1)

        return carry

    lax.fori_loop(0, CH_W, body, 0)
    store_wait(CH_W - 1)


@functools.cache
def _sc_gather():
    mesh = plsc.VectorSubcoreMesh(core_axis_name="c", subcore_axis_name="s",
                                  num_cores=NC, num_subcores=NS)
    return pl.kernel(
        _sc_gather_body,
        out_type=jax.ShapeDtypeStruct((TOK_SEG, D), jnp.float32),
        mesh=mesh,
        scratch_types=[
            pltpu.VMEM((CH_PAD, CHUNK), jnp.int32),
            pltpu.VMEM((NBUF, CHUNK, D), jnp.float32),
            pltpu.SemaphoreType.DMA,
            pltpu.SemaphoreType.DMA,
        ],
    )

L_BLK = 5
A_BLK = 1024
TC_BLOCK = L_BLK * A_BLK


def _tc_body(x_ref, g_ref, wa_ref, ba_ref, wt_ref, bt_ref, scale_ref,
             out_ref, m_ref, uv_ref):
    @pl.when((pl.program_id(0) == 0) & (pl.program_id(1) == 0))
    def _():
        wa = wa_ref[...]
        wt = wt_ref[...]
        m_ref[...] = lax.dot_general(
            wa, wt, (((0,), (0,)), ((), ())),
            preferred_element_type=jnp.float32)
        uv_ref[0:1, :] = lax.dot_general(
            bt_ref[...], wa, (((1,), (0,)), ((), ())),
            preferred_element_type=jnp.float32)
        uv_ref[1:2, :] = lax.dot_general(
            ba_ref[...], wt, (((1,), (0,)), ((), ())),
            preferred_element_type=jnp.float32)

    x = x_ref[...].reshape(TC_BLOCK, D)
    g = g_ref[...].reshape(TC_BLOCK, D)
    z = lax.dot_general(x, m_ref[...], (((1,), (0,)), ((), ())),
                        preferred_element_type=jnp.float32)
    z = z + uv_ref[1:2, :]
    dots = jnp.sum(z * g, axis=1)
    xu = jnp.sum(x * uv_ref[0:1, :], axis=1)
    c = jnp.sum(ba_ref[...] * bt_ref[...])
    out_ref[...] = ((dots + xu + c) * scale_ref[0, 0]).reshape(
        L_BLK, A_BLK // 128, 128)


def _tc_call(seg, x3, g3, Wa, ba2, Wt, bt2, scale2):
    l0 = seg * (L_SEG // L_BLK)
    return pl.pallas_call(
        _tc_body,
        grid=(L_SEG // L_BLK, B // A_BLK),
        in_specs=[
            pl.BlockSpec((L_BLK, A_BLK, D), lambda i, j: (l0 + i, j, 0)),
            pl.BlockSpec((L_BLK, A_BLK, D), lambda i, j: (i, j, 0)),
            pl.BlockSpec((D, D), lambda i, j: (0, 0)),
            pl.BlockSpec((1, D), lambda i, j: (0, 0)),
            pl.BlockSpec((D, D), lambda i, j: (0, 0)),
            pl.BlockSpec((1, D), lambda i, j: (0, 0)),
            pl.BlockSpec(memory_space=pltpu.SMEM),
        ],
        out_specs=pl.BlockSpec((L_BLK, A_BLK // 128, 128),
                               lambda i, j: (i, j, 0)),
        out_shape=jax.ShapeDtypeStruct((L_SEG, B // 128, 128), jnp.float32),
        scratch_shapes=[
            pltpu.VMEM((D, D), jnp.float32),
            pltpu.VMEM((2, D), jnp.float32),
        ],
    )(x3, g3, Wa, ba2, Wt, bt2, scale2)


def kernel(actor_emb, topic_ids, Wa, ba, table, Wt, bt, scale):
    x3 = jnp.transpose(actor_emb, (1, 0, 2))
    ids_lm = jnp.transpose(topic_ids, (1, 0)).astype(jnp.int32)

    idx4d = ids_lm.reshape(SEG, NW, CH_W, CHUNK)
    idx_pad = jnp.pad(idx4d, ((0, 0), (0, 0), (0, CH_PAD - CH_W), (0, 0)))
    idx_pad = idx_pad.reshape(SEG, NW * CH_PAD, CHUNK)

    ba2 = ba.reshape(1, D)
    bt2 = bt.reshape(1, D)
    scale2 = scale.reshape(1, 1)

    outs = []
    for s in range(SEG):
        g3 = _sc_gather()(table, idx_pad[s]).reshape(L_SEG, B, D)
        outs.append(_tc_call(s, x3, g3, Wa, ba2, Wt, bt2, scale2))
    out3 = jnp.concatenate(outs, axis=0)
    return jnp.transpose(out3.reshape(L, B), (1, 0))


# speedup vs baseline: 1.9089x; 1.1193x over previous
import functools

import jax
import jax.numpy as jnp
from jax import lax
from jax.experimental import pallas as pl
from jax.experimental.pallas import tpu as pltpu
from jax.experimental.pallas import tpu_sc as plsc

D = 128
B = 4096
L = 50
N_TOK = B * L
TOPIC_ROWS = 100001

SEG = 2
L_SEG = L // SEG
TOK_SEG = L_SEG * B

NC = 2
NS = 16
NW = NC * NS
CHUNK = 128
CH_W = TOK_SEG // (NW * CHUNK)
assert CH_W * NW * CHUNK == TOK_SEG
CH_PAD = (CH_W + 7) // 8 * 8

NBUF = 4


def _sc_gather_body(table_hbm, idx_hbm, out_hbm, idx_v, rows_v, gsem, ssem):
    wid = lax.axis_index("s") * NC + lax.axis_index("c")
    row0 = wid * CH_W

    pltpu.sync_copy(idx_hbm.at[pl.ds(wid * CH_PAD, CH_PAD)], idx_v)

    def gather(j):
        return pltpu.async_copy(
            table_hbm.at[idx_v.at[j]], rows_v.at[j % NBUF], gsem)

    def gather_wait(j):
        pltpu.make_async_copy(
            table_hbm.at[idx_v.at[j]], rows_v.at[j % NBUF], gsem).wait()

    def store(j):
        return pltpu.async_copy(
            rows_v.at[j % NBUF], out_hbm.at[pl.ds((row0 + j) * CHUNK, CHUNK)],
            ssem)

    def store_wait(j):
        pltpu.make_async_copy(
            rows_v.at[j % NBUF], out_hbm.at[pl.ds((row0 + j) * CHUNK, CHUNK)],
            ssem).wait()

    for j in range(NBUF - 1):
        gather(j)

    def body(j, carry):
        gather_wait(j)
        store(j)

        @pl.when(j >= 1)
        def _():
            store_wait(j - 1)

        @pl.when(j + NBUF - 1 < CH_W)
        def _():
            gather(j + NBUF - 1)

        return carry

    lax.fori_loop(0, CH_W, body, 0)
    store_wait(CH_W - 1)


@functools.cache
def _sc_gather():
    mesh = plsc.VectorSubcoreMesh(core_axis_name="c", subcore_axis_name="s",
                                  num_cores=NC, num_subcores=NS)
    return pl.kernel(
        _sc_gather_body,
        out_type=jax.ShapeDtypeStruct((TOK_SEG, D), jnp.float32),
        mesh=mesh,
        scratch_types=[
            pltpu.VMEM((CH_PAD, CHUNK), jnp.int32),
            pltpu.VMEM((NBUF, CHUNK, D), jnp.float32),
            pltpu.SemaphoreType.DMA,
            pltpu.SemaphoreType.DMA,
        ],
    )

A_BLK = 512
TC_BLOCK = L_SEG * A_BLK


def _tc_body(x_ref, g_ref, wa_ref, ba_ref, wt_ref, bt_ref, scale_ref,
             out_ref, m_ref, uv_ref):
    @pl.when(pl.program_id(0) == 0)
    def _():
        wa = wa_ref[...]
        wt = wt_ref[...]
        m_ref[...] = lax.dot_general(
            wa, wt, (((0,), (0,)), ((), ())),
            preferred_element_type=jnp.float32)
        uv_ref[0:1, :] = lax.dot_general(
            bt_ref[...], wa, (((1,), (0,)), ((), ())),
            preferred_element_type=jnp.float32)
        uv_ref[1:2, :] = lax.dot_general(
            ba_ref[...], wt, (((1,), (0,)), ((), ())),
            preferred_element_type=jnp.float32)

    x = x_ref[...].reshape(TC_BLOCK, D)
    g = g_ref[...].reshape(TC_BLOCK, D)
    z = lax.dot_general(x, m_ref[...], (((1,), (0,)), ((), ())),
                        preferred_element_type=jnp.float32)
    q = (z + uv_ref[1:2, :]) * g + x * uv_ref[0:1, :]
    dots = jnp.sum(q.reshape(L_SEG, A_BLK, D), axis=2)
    c = jnp.sum(ba_ref[...] * bt_ref[...])
    out_ref[...] = (dots + c) * scale_ref[0, 0]


def _tc_call(seg, x3, g3, Wa, ba2, Wt, bt2, scale2):
    return pl.pallas_call(
        _tc_body,
        grid=(B // A_BLK,),
        in_specs=[
            pl.BlockSpec((L_SEG, A_BLK, D), lambda j: (seg, j, 0)),
            pl.BlockSpec((L_SEG, A_BLK, D), lambda j: (0, j, 0)),
            pl.BlockSpec((D, D), lambda j: (0, 0)),
            pl.BlockSpec((1, D), lambda j: (0, 0)),
            pl.BlockSpec((D, D), lambda j: (0, 0)),
            pl.BlockSpec((1, D), lambda j: (0, 0)),
            pl.BlockSpec(memory_space=pltpu.SMEM),
        ],
        out_specs=pl.BlockSpec((L_SEG, A_BLK), lambda j: (0, j)),
        out_shape=jax.ShapeDtypeStruct((L_SEG, B), jnp.float32),
        scratch_shapes=[
            pltpu.VMEM((D, D), jnp.float32),
            pltpu.VMEM((2, D), jnp.float32),
        ],
    )(x3, g3, Wa, ba2, Wt, bt2, scale2)


def kernel(actor_emb, topic_ids, Wa, ba, table, Wt, bt, scale):
    x3 = jnp.transpose(actor_emb, (1, 0, 2))
    ids_lm = jnp.transpose(topic_ids, (1, 0)).astype(jnp.int32)

    idx4d = ids_lm.reshape(SEG, NW, CH_W, CHUNK)
    idx_pad = jnp.pad(idx4d, ((0, 0), (0, 0), (0, CH_PAD - CH_W), (0, 0)))
    idx_pad = idx_pad.reshape(SEG, NW * CH_PAD, CHUNK)

    ba2 = ba.reshape(1, D)
    bt2 = bt.reshape(1, D)
    scale2 = scale.reshape(1, 1)

    outs = []
    for s in range(SEG):
        g3 = _sc_gather()(table, idx_pad[s]).reshape(L_SEG, B, D)
        outs.append(_tc_call(s, x3, g3, Wa, ba2, Wt, bt2, scale2))
    out2 = jnp.concatenate(outs, axis=0)
    return jnp.transpose(out2, (1, 0))
